# Initial kernel scaffold; baseline (speedup 1.0000x reference)
import jax
import jax.numpy as jnp
from jax.experimental import pallas as pl


def kernel(x, weights):
    raise NotImplementedError("write your pallas kernel here")


# speedup vs baseline: 3.9307x; 3.9307x over previous
import functools

import jax
import jax.numpy as jnp
from jax import lax
from jax.experimental import pallas as pl
from jax.experimental.pallas import tpu as pltpu
from jax.experimental.pallas import tpu_sc as plsc

_NC = 2
_NS = 16
_NW = _NC * _NS


def _positional_lookup(table, idx, *, chunk):
    n_rows, d = table.shape
    b = idx.shape[0]
    bpw = b // _NW
    n_chunks = bpw // chunk
    mesh = plsc.VectorSubcoreMesh(core_axis_name="c", subcore_axis_name="s")

    @functools.partial(
        pl.kernel,
        mesh=mesh,
        out_type=jax.ShapeDtypeStruct((b, d), jnp.float32),
        scratch_types=[
            pltpu.VMEM((bpw,), jnp.int32),
            pltpu.VMEM((chunk, d), jnp.float32),
            pltpu.SemaphoreType.DMA,
        ],
    )
    def k(idx_hbm, tab_hbm, out_hbm, idx_v, rows_v, gsem):
        wid = lax.axis_index("s") * _NC + lax.axis_index("c")
        base = wid * bpw
        pltpu.sync_copy(idx_hbm.at[pl.ds(base, bpw)], idx_v)

        def body(c, carry):
            off = c * chunk
            pltpu.async_copy(
                tab_hbm.at[idx_v.at[pl.ds(off, chunk)]], rows_v, gsem
            ).wait()
            pltpu.sync_copy(rows_v, out_hbm.at[pl.ds(base + off, chunk)])
            return carry

        lax.fori_loop(0, n_chunks, body, 0)

    return k(idx, table)


def kernel(x, weights):
    n_rows = weights.shape[0]
    d = weights.shape[1] * weights.shape[2]
    table = weights.reshape(n_rows, d)
    out = _positional_lookup(table, x, chunk=16)
    return out


# speedup vs baseline: 4.0943x; 1.0416x over previous
import functools

import jax
import jax.numpy as jnp
from jax import lax
from jax.experimental import pallas as pl
from jax.experimental.pallas import tpu as pltpu
from jax.experimental.pallas import tpu_sc as plsc

_NC = 2
_NS = 16
_NW = _NC * _NS


def _positional_lookup(table, idx, *, chunk):
    n_rows, d = table.shape
    b = idx.shape[0]
    bpw = b // _NW
    n_chunks = bpw // chunk
    mesh = plsc.VectorSubcoreMesh(core_axis_name="c", subcore_axis_name="s")

    @functools.partial(
        pl.kernel,
        mesh=mesh,
        out_type=jax.ShapeDtypeStruct((b, d), jnp.float32),
        scratch_types=[
            pltpu.VMEM((bpw,), jnp.int32),
            pltpu.VMEM((2, chunk, d), jnp.float32),
            pltpu.SemaphoreType.DMA,
            pltpu.SemaphoreType.DMA,
            pltpu.SemaphoreType.DMA,
            pltpu.SemaphoreType.DMA,
        ],
    )
    def k(idx_hbm, tab_hbm, out_hbm, idx_v, rows_v, g0, g1, w0, w1):
        gs = (g0, g1)
        ws = (w0, w1)
        wid = lax.axis_index("s") * _NC + lax.axis_index("c")
        base = wid * bpw
        pltpu.sync_copy(idx_hbm.at[pl.ds(base, bpw)], idx_v)

        def gather(c, j):
            return pltpu.make_async_copy(
                tab_hbm.at[idx_v.at[pl.ds(c * chunk, chunk)]],
                rows_v.at[j],
                gs[j],
            )

        def write(c, j):
            return pltpu.make_async_copy(
                rows_v.at[j],
                out_hbm.at[pl.ds(base + c * chunk, chunk)],
                ws[j],
            )

        gather(0, 0).start()

        def body(i, carry):
            for j in range(2):
                c = 2 * i + j
                gather(c, j).wait()
                write(c, j).start()
                cn = c + 1

                @pl.when(cn < n_chunks)
                def _():
                    @pl.when(cn >= 2)
                    def _():
                        write(cn - 2, 1 - j).wait()

                    gather(cn, 1 - j).start()

            return carry

        lax.fori_loop(0, n_chunks // 2, body, 0)
        write(n_chunks - 2, 0).wait()
        write(n_chunks - 1, 1).wait()

    return k(idx, table)


def kernel(x, weights):
    n_rows = weights.shape[0]
    d = weights.shape[1] * weights.shape[2]
    table = weights.reshape(n_rows, d)
    out = _positional_lookup(table, x, chunk=8)
    return out
